# baseline (speedup 1.0000x reference)
import functools

import jax
import jax.numpy as jnp
from jax import lax
from jax.experimental import pallas as pl
from jax.experimental.pallas import tpu as pltpu
from jax.experimental.pallas import tpu_sc as plsc

B = 16384
DIM = 32
RAW_DIM = 96
OUT_DIM = 64
NC = 2
NS = 16
NW = NC * NS
BPW = B // NW

BV = 4096
TC_SPLIT = BV * 158


def _mesh():
    return plsc.VectorSubcoreMesh(core_axis_name="c", subcore_axis_name="s")


def _tc_pack(pt, nblk):
    def body(x_ref, o_ref, xt_ref):
        xt_ref[...] = x_ref[...].T
        for s in range(4):
            o_ref[:, s * DIM : (s + 1) * DIM] = xt_ref[s :: 4, :]

    return pl.pallas_call(
        body,
        grid=(nblk,),
        in_specs=[pl.BlockSpec((DIM, BV), lambda i: (0, i))],
        out_specs=pl.BlockSpec((BV // 4, 4 * DIM), lambda i: (i, 0)),
        out_shape=jax.ShapeDtypeStruct((nblk * BV // 4, 4 * DIM), jnp.float32),
        scratch_shapes=[pltpu.VMEM((BV, DIM), jnp.float32)],
        compiler_params=pltpu.CompilerParams(
            dimension_semantics=("parallel",)
        ),
    )(pt)


def _sc_pack(pt_cat, pt_brand, pt_store):
    v_cat = pt_cat.shape[1]
    cov_cat = (v_cat // 128) * 128
    span_cat = cov_cat - TC_SPLIT
    out_specs = [
        (span_cat // 4, 1, TC_SPLIT),
        ((pt_brand.shape[1] // 128) * 32, 1, 0),
        ((pt_store.shape[1] // 128) * 32, 1, 0),
    ]

    @functools.partial(
        pl.kernel,
        mesh=_mesh(),
        out_type=[
            jax.ShapeDtypeStruct((rows, 4 * DIM), jnp.float32)
            for rows, _, _ in out_specs
        ],
        compiler_params=pltpu.CompilerParams(needs_layout_passes=False),
        scratch_types=[
            pltpu.VMEM((DIM, 256), jnp.float32),
            pltpu.VMEM((DIM, 256), jnp.float32),
            pltpu.VMEM((64, 4 * DIM), jnp.float32),
            pltpu.VMEM((64, 4 * DIM), jnp.float32),
            pltpu.SemaphoreType.DMA,
            pltpu.SemaphoreType.DMA,
            pltpu.SemaphoreType.DMA,
            pltpu.SemaphoreType.DMA,
        ],
    )
    def k(tc_, tb_, ts_, oc, ob, osr, w0, w1, p0, p1, si0, si1, so0, so1):
        wid = lax.axis_index("s") * NC + lax.axis_index("c")
        iota = lax.iota(jnp.int32, 16)

        def shuffle(win, pout, sb):
            for sub in range(sb):
                for j0 in range(0, 32, 4):
                    vals = [
                        plsc.load_gather(
                            win,
                            [
                                iota + 16 * (q % 2),
                                jnp.full(
                                    (16,), 128 * sub + 4 * j + q // 2, jnp.int32
                                ),
                            ],
                        )
                        for j in range(j0, j0 + 4)
                        for q in range(8)
                    ]
                    for k2, j in enumerate(range(j0, j0 + 4)):
                        for q in range(8):
                            pout[32 * sub + j, pl.ds(16 * q, 16)] = vals[
                                k2 * 8 + q
                            ]

        for (t_hbm, o_hbm), (rows, sb, base) in zip(
            ((tc_, oc), (tb_, ob), (ts_, osr)), out_specs
        ):
            ntot = (rows * 4) // (128 * sb)
            per = -(-ntot // NW)
            lo = wid * per
            n = jnp.clip(ntot - lo, 0, per)

            def issue_in(i, buf, sem):
                src = pl.multiple_of(base + (lo + i) * 128 * sb, 128)
                pltpu.async_copy(
                    t_hbm.at[:, pl.ds(src, 128 * sb)],
                    buf.at[:, pl.ds(0, 128 * sb)],
                    sem,
                )

            def wait_in(buf, sem):
                pltpu.make_async_copy(
                    t_hbm.at[:, pl.ds(0, 128 * sb)],
                    buf.at[:, pl.ds(0, 128 * sb)],
                    sem,
                ).wait()

            def issue_out(i, buf, sem):
                dst = pl.multiple_of((lo + i) * 32 * sb, 32)
                pltpu.async_copy(
                    buf.at[pl.ds(0, 32 * sb)],
                    o_hbm.at[pl.ds(dst, 32 * sb)],
                    sem,
                )

            def wait_out(buf, sem):
                pltpu.make_async_copy(
                    buf.at[pl.ds(0, 32 * sb)],
                    o_hbm.at[pl.ds(0, 32 * sb)],
                    sem,
                ).wait()

            @pl.when(n > 0)
            def _():
                issue_in(0, w0, si0)

            def slot(i, w, p, si_a, si_b, w_other, so):
                @pl.when(i < n)
                def _():
                    @pl.when(i + 1 < n)
                    def _():
                        issue_in(i + 1, w_other, si_b)

                    wait_in(w, si_a)

                    @pl.when(i >= 2)
                    def _():
                        wait_out(p, so)

                    shuffle(w, p, sb)
                    issue_out(i, p, so)

            @pl.loop(0, per, step=2)
            def _(i):
                slot(i, w0, p0, si0, si1, w1, so0)
                slot(i + 1, w1, p1, si1, si0, w0, so1)

            @pl.when(n >= 1)
            def _():
                wait_out(p0, so0)

            @pl.when(n >= 2)
            def _():
                wait_out(p1, so1)

    return k(pt_cat, pt_brand, pt_store)


def _sc_gather(idxs, tables):
    @functools.partial(
        pl.kernel,
        mesh=_mesh(),
        out_type=jax.ShapeDtypeStruct((B, 4 * 4 * DIM), jnp.float32),
        scratch_types=[
            pltpu.VMEM((BPW,), jnp.int32),
            pltpu.VMEM((BPW, 4 * DIM), jnp.float32),
            pltpu.SemaphoreType.DMA,
        ],
    )
    def k(i0, i1, i2, i3, t0, t1, t2, t3, xo, idx_v, rows_v, sem):
        wid = lax.axis_index("s") * NC + lax.axis_index("c")
        base = wid * BPW
        for t, (i_hbm, t_hbm) in enumerate(
            ((i0, t0), (i1, t1), (i2, t2), (i3, t3))
        ):
            pltpu.sync_copy(i_hbm.at[pl.ds(base, BPW)], idx_v)
            pltpu.async_copy(t_hbm.at[idx_v], rows_v, sem).wait()
            pltpu.sync_copy(
                rows_v, xo.at[pl.ds(base, BPW), pl.ds(t * 4 * DIM, 4 * DIM)]
            )

    return k(*idxs, *tables)


BM = 2048


def _tc_project(x, offs, tails, covs, Wt, b2):
    lane_group = 4 * DIM

    def body(x_ref, o_ref, tc_ref, tb_ref, ts_ref, w_ref, bias_ref, out_ref):
        def select(t, grp_idx):
            off = jnp.broadcast_to(o_ref[:, t : t + 1], (BM, lane_group))
            grp = lax.broadcasted_iota(jnp.int32, (BM, lane_group), 1) // DIM
            xm = jnp.where(
                grp == off,
                x_ref[:, grp_idx * lane_group : (grp_idx + 1) * lane_group],
                0.0,
            )
            return (
                xm[:, 0:DIM]
                + xm[:, DIM : 2 * DIM]
                + xm[:, 2 * DIM : 3 * DIM]
                + xm[:, 3 * DIM : 4 * DIM]
            )

        def tail_fix(t, t_ref, g):
            tv = t_ref.shape[0]
            idx = o_ref[:, 4 + t : 5 + t]
            rel = jnp.broadcast_to(idx - covs[t], (BM, tv))
            oh = jnp.where(
                lax.broadcasted_iota(jnp.int32, (BM, tv), 1) == rel, 1.0, 0.0
            )
            y_tail = jnp.dot(oh, t_ref[...], preferred_element_type=jnp.float32)
            ok = jnp.where(
                jnp.broadcast_to(idx, (BM, DIM)) < covs[t], 1.0, 0.0
            )
            return g * ok + y_tail

        g_lo = select(0, 0)
        g_hi = select(0, 1)
        use_lo = jnp.broadcast_to(o_ref[:, 4:5], (BM, DIM)) < TC_SPLIT
        g_cat = jnp.where(use_lo, g_lo, g_hi)
        sel = [
            tail_fix(0, tc_ref, g_cat),
            tail_fix(1, tb_ref, select(1, 2)),
            tail_fix(2, ts_ref, select(2, 3)),
        ]
        xs = jnp.concatenate(sel, axis=1)
        y = jnp.dot(xs, w_ref[...], preferred_element_type=jnp.float32)
        out_ref[...] = jnp.maximum(y + bias_ref[...], 0.0)

    tv_c, tv_b, tv_s = (t.shape[0] for t in tails)
    return pl.pallas_call(
        body,
        grid=(B // BM,),
        in_specs=[
            pl.BlockSpec((BM, 4 * lane_group), lambda i: (i, 0)),
            pl.BlockSpec((BM, 8), lambda i: (i, 0)),
            pl.BlockSpec((tv_c, DIM), lambda i: (0, 0)),
            pl.BlockSpec((tv_b, DIM), lambda i: (0, 0)),
            pl.BlockSpec((tv_s, DIM), lambda i: (0, 0)),
            pl.BlockSpec((RAW_DIM, OUT_DIM), lambda i: (0, 0)),
            pl.BlockSpec((1, OUT_DIM), lambda i: (0, 0)),
        ],
        out_specs=pl.BlockSpec((BM, OUT_DIM), lambda i: (i, 0)),
        out_shape=jax.ShapeDtypeStruct((B, OUT_DIM), jnp.float32),
        compiler_params=pltpu.CompilerParams(
            dimension_semantics=("parallel",)
        ),
    )(x, offs, *tails, Wt, b2)


def kernel(category, brand, store, emb_category, emb_brand, emb_store, W, b):
    ci = category.astype(jnp.int32)
    bi = brand.astype(jnp.int32)
    si = store.astype(jnp.int32)
    covs = tuple((e.shape[0] // 128) * 128
                 for e in (emb_category, emb_brand, emb_store))
    p_cat_lo = _tc_pack(emb_category.T, TC_SPLIT // BV)
    p_cat_hi, p_brand, p_store = _sc_pack(
        emb_category.T, emb_brand.T, emb_store.T
    )
    i_lo = jnp.minimum(ci >> 2, TC_SPLIT // 4 - 1)
    i_hi = jnp.clip((ci - TC_SPLIT) >> 2, 0, (covs[0] - TC_SPLIT) // 4 - 1)
    i_b = jnp.minimum(bi >> 2, covs[1] // 4 - 1)
    i_s = jnp.minimum(si >> 2, covs[2] // 4 - 1)
    x = _sc_gather(
        (i_lo, i_hi, i_b, i_s), (p_cat_lo, p_cat_hi, p_brand, p_store)
    )
    z = jnp.zeros_like(ci)
    offs = jnp.stack([ci & 3, bi & 3, si & 3, z, ci, bi, si, z], axis=1)
    tails = (
        emb_category[covs[0] :],
        emb_brand[covs[1] :],
        emb_store[covs[2] :],
    )
    Wt = W.T
    b2 = b.reshape(1, OUT_DIM)
    return _tc_project(x, offs, tails, covs, Wt, b2)


# speedup vs baseline: 2.2884x; 2.2884x over previous
import functools

import jax
import jax.numpy as jnp
from jax import lax
from jax.experimental import pallas as pl
from jax.experimental.pallas import tpu as pltpu
from jax.experimental.pallas import tpu_sc as plsc

B = 16384
DIM = 32
RAW_DIM = 96
OUT_DIM = 64
NC = 2
NS = 16
NW = NC * NS
BPW = B // NW

BV = 4096


def _tc_pack(pt):
    v = pt.shape[1]
    nblk = (v + BV - 1) // BV

    def body(x_ref, o_ref, xt_ref):
        xt_ref[...] = x_ref[...].T
        for s in range(4):
            o_ref[:, s * DIM : (s + 1) * DIM] = xt_ref[s :: 4, :]

    return pl.pallas_call(
        body,
        grid=(nblk,),
        in_specs=[pl.BlockSpec((DIM, BV), lambda i: (0, i))],
        out_specs=pl.BlockSpec((BV // 4, 4 * DIM), lambda i: (i, 0)),
        out_shape=jax.ShapeDtypeStruct((v // 4, 4 * DIM), jnp.float32),
        scratch_shapes=[pltpu.VMEM((BV, DIM), jnp.float32)],
        compiler_params=pltpu.CompilerParams(
            dimension_semantics=("parallel",)
        ),
    )(pt)


def _sc_gather3(i4_cat, i4_brand, i4_store, p_cat, p_brand, p_store):
    mesh = plsc.VectorSubcoreMesh(core_axis_name="c", subcore_axis_name="s")

    @functools.partial(
        pl.kernel,
        mesh=mesh,
        out_type=jax.ShapeDtypeStruct((B, 3 * 4 * DIM), jnp.float32),
        scratch_types=[
            pltpu.VMEM((BPW,), jnp.int32),
            pltpu.VMEM((BPW, 4 * DIM), jnp.float32),
            pltpu.SemaphoreType.DMA,
        ],
    )
    def k(ci, bi, si, pc, pb, ps, xo, idx_v, rows_v, sem):
        wid = lax.axis_index("s") * NC + lax.axis_index("c")
        base = wid * BPW
        for t, (i_hbm, t_hbm) in enumerate(
            ((ci, pc), (bi, pb), (si, ps))
        ):
            pltpu.sync_copy(i_hbm.at[pl.ds(base, BPW)], idx_v)
            pltpu.async_copy(t_hbm.at[idx_v], rows_v, sem).wait()
            pltpu.sync_copy(
                rows_v, xo.at[pl.ds(base, BPW), pl.ds(t * 4 * DIM, 4 * DIM)]
            )

    return k(i4_cat, i4_brand, i4_store, p_cat, p_brand, p_store)


BM = 2048


def _tc_project(x, offs, Wt, b2):
    lane_group = 4 * DIM

    def body(x_ref, o_ref, w_ref, bias_ref, out_ref):
        sel = []
        for t in range(3):
            off = jnp.broadcast_to(o_ref[:, t : t + 1], (BM, lane_group))
            grp = lax.broadcasted_iota(jnp.int32, (BM, lane_group), 1) // DIM
            xm = jnp.where(
                grp == off, x_ref[:, t * lane_group : (t + 1) * lane_group], 0.0
            )
            sel.append(
                xm[:, 0:DIM]
                + xm[:, DIM : 2 * DIM]
                + xm[:, 2 * DIM : 3 * DIM]
                + xm[:, 3 * DIM : 4 * DIM]
            )
        xs = jnp.concatenate(sel, axis=1)
        y = jnp.dot(xs, w_ref[...], preferred_element_type=jnp.float32)
        out_ref[...] = jnp.maximum(y + bias_ref[...], 0.0)

    return pl.pallas_call(
        body,
        grid=(B // BM,),
        in_specs=[
            pl.BlockSpec((BM, 3 * 4 * DIM), lambda i: (i, 0)),
            pl.BlockSpec((BM, 4), lambda i: (i, 0)),
            pl.BlockSpec((RAW_DIM, OUT_DIM), lambda i: (0, 0)),
            pl.BlockSpec((1, OUT_DIM), lambda i: (0, 0)),
        ],
        out_specs=pl.BlockSpec((BM, OUT_DIM), lambda i: (i, 0)),
        out_shape=jax.ShapeDtypeStruct((B, OUT_DIM), jnp.float32),
        compiler_params=pltpu.CompilerParams(
            dimension_semantics=("parallel",)
        ),
    )(x, offs, Wt, b2)


def kernel(category, brand, store, emb_category, emb_brand, emb_store, W, b):
    ci = category.astype(jnp.int32)
    bi = brand.astype(jnp.int32)
    si = store.astype(jnp.int32)
    p_cat = _tc_pack(emb_category.T)
    p_brand = emb_brand.reshape(-1, 4 * DIM)
    p_store = emb_store.reshape(-1, 4 * DIM)
    x = _sc_gather3(ci >> 2, bi >> 2, si >> 2, p_cat, p_brand, p_store)
    offs = jnp.stack([ci & 3, bi & 3, si & 3, jnp.zeros_like(ci)], axis=1)
    Wt = W.T
    b2 = b.reshape(1, OUT_DIM)
    return _tc_project(x, offs, Wt, b2)


# speedup vs baseline: 2.4609x; 1.0754x over previous
import functools

import jax
import jax.numpy as jnp
from jax import lax
from jax.experimental import pallas as pl
from jax.experimental.pallas import tpu as pltpu
from jax.experimental.pallas import tpu_sc as plsc

B = 16384
DIM = 32
RAW_DIM = 96
OUT_DIM = 64
NC = 2
NS = 16
NW = NC * NS
BPW = B // NW

BV = 8192


def _tc_pack(pt):
    v = pt.shape[1]
    nblk = (v + BV - 1) // BV

    def body(x_ref, o_ref, xt_ref):
        xt_ref[...] = x_ref[...].T
        for s in range(4):
            o_ref[:, s * DIM : (s + 1) * DIM] = xt_ref[s :: 4, :]

    return pl.pallas_call(
        body,
        grid=(nblk,),
        in_specs=[pl.BlockSpec((DIM, BV), lambda i: (0, i))],
        out_specs=pl.BlockSpec((BV // 4, 4 * DIM), lambda i: (i, 0)),
        out_shape=jax.ShapeDtypeStruct((v // 4, 4 * DIM), jnp.float32),
        scratch_shapes=[pltpu.VMEM((BV, DIM), jnp.float32)],
        compiler_params=pltpu.CompilerParams(
            dimension_semantics=("parallel",)
        ),
    )(pt)


def _sc_gather3(i4_cat, i4_brand, i4_store, p_cat, p_brand, p_store):
    mesh = plsc.VectorSubcoreMesh(core_axis_name="c", subcore_axis_name="s")

    @functools.partial(
        pl.kernel,
        mesh=mesh,
        out_type=jax.ShapeDtypeStruct((B, 3 * 4 * DIM), jnp.float32),
        scratch_types=[
            pltpu.VMEM((BPW,), jnp.int32),
            pltpu.VMEM((BPW, 4 * DIM), jnp.float32),
            pltpu.SemaphoreType.DMA,
        ],
    )
    def k(ci, bi, si, pc, pb, ps, xo, idx_v, rows_v, sem):
        wid = lax.axis_index("s") * NC + lax.axis_index("c")
        base = wid * BPW
        for t, (i_hbm, t_hbm) in enumerate(
            ((ci, pc), (bi, pb), (si, ps))
        ):
            pltpu.sync_copy(i_hbm.at[pl.ds(base, BPW)], idx_v)
            pltpu.async_copy(t_hbm.at[idx_v], rows_v, sem).wait()
            pltpu.sync_copy(
                rows_v, xo.at[pl.ds(base, BPW), pl.ds(t * 4 * DIM, 4 * DIM)]
            )

    return k(i4_cat, i4_brand, i4_store, p_cat, p_brand, p_store)


BM = 2048


def _tc_project(x, offs, Wt, b2):
    lane_group = 4 * DIM

    def body(x_ref, o_ref, w_ref, bias_ref, out_ref):
        sel = []
        for t in range(3):
            off = jnp.broadcast_to(o_ref[:, t : t + 1], (BM, lane_group))
            grp = lax.broadcasted_iota(jnp.int32, (BM, lane_group), 1) // DIM
            xm = jnp.where(
                grp == off, x_ref[:, t * lane_group : (t + 1) * lane_group], 0.0
            )
            sel.append(
                xm[:, 0:DIM]
                + xm[:, DIM : 2 * DIM]
                + xm[:, 2 * DIM : 3 * DIM]
                + xm[:, 3 * DIM : 4 * DIM]
            )
        xs = jnp.concatenate(sel, axis=1)
        y = jnp.dot(xs, w_ref[...], preferred_element_type=jnp.float32)
        out_ref[...] = jnp.maximum(y + bias_ref[...], 0.0)

    return pl.pallas_call(
        body,
        grid=(B // BM,),
        in_specs=[
            pl.BlockSpec((BM, 3 * 4 * DIM), lambda i: (i, 0)),
            pl.BlockSpec((BM, 4), lambda i: (i, 0)),
            pl.BlockSpec((RAW_DIM, OUT_DIM), lambda i: (0, 0)),
            pl.BlockSpec((1, OUT_DIM), lambda i: (0, 0)),
        ],
        out_specs=pl.BlockSpec((BM, OUT_DIM), lambda i: (i, 0)),
        out_shape=jax.ShapeDtypeStruct((B, OUT_DIM), jnp.float32),
        compiler_params=pltpu.CompilerParams(
            dimension_semantics=("parallel",)
        ),
    )(x, offs, Wt, b2)


def kernel(category, brand, store, emb_category, emb_brand, emb_store, W, b):
    ci = category.astype(jnp.int32)
    bi = brand.astype(jnp.int32)
    si = store.astype(jnp.int32)
    p_cat = _tc_pack(emb_category.T)
    p_brand = emb_brand.reshape(-1, 4 * DIM)
    p_store = emb_store.reshape(-1, 4 * DIM)
    x = _sc_gather3(ci >> 2, bi >> 2, si >> 2, p_cat, p_brand, p_store)
    offs = jnp.stack([ci & 3, bi & 3, si & 3, jnp.zeros_like(ci)], axis=1)
    Wt = W.T
    b2 = b.reshape(1, OUT_DIM)
    return _tc_project(x, offs, Wt, b2)


# speedup vs baseline: 2.4895x; 1.0116x over previous
import functools

import jax
import jax.numpy as jnp
from jax import lax
from jax.experimental import pallas as pl
from jax.experimental.pallas import tpu as pltpu
from jax.experimental.pallas import tpu_sc as plsc

B = 16384
DIM = 32
RAW_DIM = 96
OUT_DIM = 64
NC = 2
NS = 16
NW = NC * NS
BPW = B // NW

BV = 16384


def _tc_pack(pt):
    v = pt.shape[1]
    nblk = (v + BV - 1) // BV

    def body(x_ref, o_ref, xt_ref):
        xt_ref[...] = x_ref[...].T
        for s in range(4):
            o_ref[:, s * DIM : (s + 1) * DIM] = xt_ref[s :: 4, :]

    return pl.pallas_call(
        body,
        grid=(nblk,),
        in_specs=[pl.BlockSpec((DIM, BV), lambda i: (0, i))],
        out_specs=pl.BlockSpec((BV // 4, 4 * DIM), lambda i: (i, 0)),
        out_shape=jax.ShapeDtypeStruct((v // 4, 4 * DIM), jnp.float32),
        scratch_shapes=[pltpu.VMEM((BV, DIM), jnp.float32)],
        compiler_params=pltpu.CompilerParams(
            dimension_semantics=("parallel",)
        ),
    )(pt)


def _sc_gather3(i4_cat, i4_brand, i4_store, p_cat, p_brand, p_store):
    mesh = plsc.VectorSubcoreMesh(core_axis_name="c", subcore_axis_name="s")

    @functools.partial(
        pl.kernel,
        mesh=mesh,
        out_type=jax.ShapeDtypeStruct((B, 3 * 4 * DIM), jnp.float32),
        scratch_types=[
            pltpu.VMEM((BPW,), jnp.int32),
            pltpu.VMEM((BPW, 4 * DIM), jnp.float32),
            pltpu.SemaphoreType.DMA,
        ],
    )
    def k(ci, bi, si, pc, pb, ps, xo, idx_v, rows_v, sem):
        wid = lax.axis_index("s") * NC + lax.axis_index("c")
        base = wid * BPW
        for t, (i_hbm, t_hbm) in enumerate(
            ((ci, pc), (bi, pb), (si, ps))
        ):
            pltpu.sync_copy(i_hbm.at[pl.ds(base, BPW)], idx_v)
            pltpu.async_copy(t_hbm.at[idx_v], rows_v, sem).wait()
            pltpu.sync_copy(
                rows_v, xo.at[pl.ds(base, BPW), pl.ds(t * 4 * DIM, 4 * DIM)]
            )

    return k(i4_cat, i4_brand, i4_store, p_cat, p_brand, p_store)


BM = 2048


def _tc_project(x, offs, Wt, b2):
    lane_group = 4 * DIM

    def body(x_ref, o_ref, w_ref, bias_ref, out_ref):
        sel = []
        for t in range(3):
            off = jnp.broadcast_to(o_ref[:, t : t + 1], (BM, lane_group))
            grp = lax.broadcasted_iota(jnp.int32, (BM, lane_group), 1) // DIM
            xm = jnp.where(
                grp == off, x_ref[:, t * lane_group : (t + 1) * lane_group], 0.0
            )
            sel.append(
                xm[:, 0:DIM]
                + xm[:, DIM : 2 * DIM]
                + xm[:, 2 * DIM : 3 * DIM]
                + xm[:, 3 * DIM : 4 * DIM]
            )
        xs = jnp.concatenate(sel, axis=1)
        y = jnp.dot(xs, w_ref[...], preferred_element_type=jnp.float32)
        out_ref[...] = jnp.maximum(y + bias_ref[...], 0.0)

    return pl.pallas_call(
        body,
        grid=(B // BM,),
        in_specs=[
            pl.BlockSpec((BM, 3 * 4 * DIM), lambda i: (i, 0)),
            pl.BlockSpec((BM, 4), lambda i: (i, 0)),
            pl.BlockSpec((RAW_DIM, OUT_DIM), lambda i: (0, 0)),
            pl.BlockSpec((1, OUT_DIM), lambda i: (0, 0)),
        ],
        out_specs=pl.BlockSpec((BM, OUT_DIM), lambda i: (i, 0)),
        out_shape=jax.ShapeDtypeStruct((B, OUT_DIM), jnp.float32),
        compiler_params=pltpu.CompilerParams(
            dimension_semantics=("parallel",)
        ),
    )(x, offs, Wt, b2)


def kernel(category, brand, store, emb_category, emb_brand, emb_store, W, b):
    ci = category.astype(jnp.int32)
    bi = brand.astype(jnp.int32)
    si = store.astype(jnp.int32)
    p_cat = _tc_pack(emb_category.T)
    p_brand = emb_brand.reshape(-1, 4 * DIM)
    p_store = emb_store.reshape(-1, 4 * DIM)
    x = _sc_gather3(ci >> 2, bi >> 2, si >> 2, p_cat, p_brand, p_store)
    offs = jnp.stack([ci & 3, bi & 3, si & 3, jnp.zeros_like(ci)], axis=1)
    Wt = W.T
    b2 = b.reshape(1, OUT_DIM)
    return _tc_project(x, offs, Wt, b2)


# speedup vs baseline: 2.4980x; 1.0034x over previous
import functools

import jax
import jax.numpy as jnp
from jax import lax
from jax.experimental import pallas as pl
from jax.experimental.pallas import tpu as pltpu
from jax.experimental.pallas import tpu_sc as plsc

B = 16384
DIM = 32
RAW_DIM = 96
OUT_DIM = 64
NC = 2
NS = 16
NW = NC * NS
BPW = B // NW

BV = 32768


def _tc_pack(pt):
    v = pt.shape[1]
    nblk = (v + BV - 1) // BV

    def body(x_ref, o_ref, xt_ref):
        xt_ref[...] = x_ref[...].T
        for s in range(4):
            o_ref[:, s * DIM : (s + 1) * DIM] = xt_ref[s :: 4, :]

    return pl.pallas_call(
        body,
        grid=(nblk,),
        in_specs=[pl.BlockSpec((DIM, BV), lambda i: (0, i))],
        out_specs=pl.BlockSpec((BV // 4, 4 * DIM), lambda i: (i, 0)),
        out_shape=jax.ShapeDtypeStruct((v // 4, 4 * DIM), jnp.float32),
        scratch_shapes=[pltpu.VMEM((BV, DIM), jnp.float32)],
        compiler_params=pltpu.CompilerParams(
            dimension_semantics=("parallel",)
        ),
    )(pt)


def _sc_gather3(i4_cat, i4_brand, i4_store, p_cat, p_brand, p_store):
    mesh = plsc.VectorSubcoreMesh(core_axis_name="c", subcore_axis_name="s")

    @functools.partial(
        pl.kernel,
        mesh=mesh,
        out_type=jax.ShapeDtypeStruct((B, 3 * 4 * DIM), jnp.float32),
        scratch_types=[
            pltpu.VMEM((BPW,), jnp.int32),
            pltpu.VMEM((BPW, 4 * DIM), jnp.float32),
            pltpu.SemaphoreType.DMA,
        ],
    )
    def k(ci, bi, si, pc, pb, ps, xo, idx_v, rows_v, sem):
        wid = lax.axis_index("s") * NC + lax.axis_index("c")
        base = wid * BPW
        for t, (i_hbm, t_hbm) in enumerate(
            ((ci, pc), (bi, pb), (si, ps))
        ):
            pltpu.sync_copy(i_hbm.at[pl.ds(base, BPW)], idx_v)
            pltpu.async_copy(t_hbm.at[idx_v], rows_v, sem).wait()
            pltpu.sync_copy(
                rows_v, xo.at[pl.ds(base, BPW), pl.ds(t * 4 * DIM, 4 * DIM)]
            )

    return k(i4_cat, i4_brand, i4_store, p_cat, p_brand, p_store)


BM = 2048


def _tc_project(x, offs, Wt, b2):
    lane_group = 4 * DIM

    def body(x_ref, o_ref, w_ref, bias_ref, out_ref):
        sel = []
        for t in range(3):
            off = jnp.broadcast_to(o_ref[:, t : t + 1], (BM, lane_group))
            grp = lax.broadcasted_iota(jnp.int32, (BM, lane_group), 1) // DIM
            xm = jnp.where(
                grp == off, x_ref[:, t * lane_group : (t + 1) * lane_group], 0.0
            )
            sel.append(
                xm[:, 0:DIM]
                + xm[:, DIM : 2 * DIM]
                + xm[:, 2 * DIM : 3 * DIM]
                + xm[:, 3 * DIM : 4 * DIM]
            )
        xs = jnp.concatenate(sel, axis=1)
        y = jnp.dot(xs, w_ref[...], preferred_element_type=jnp.float32)
        out_ref[...] = jnp.maximum(y + bias_ref[...], 0.0)

    return pl.pallas_call(
        body,
        grid=(B // BM,),
        in_specs=[
            pl.BlockSpec((BM, 3 * 4 * DIM), lambda i: (i, 0)),
            pl.BlockSpec((BM, 4), lambda i: (i, 0)),
            pl.BlockSpec((RAW_DIM, OUT_DIM), lambda i: (0, 0)),
            pl.BlockSpec((1, OUT_DIM), lambda i: (0, 0)),
        ],
        out_specs=pl.BlockSpec((BM, OUT_DIM), lambda i: (i, 0)),
        out_shape=jax.ShapeDtypeStruct((B, OUT_DIM), jnp.float32),
        compiler_params=pltpu.CompilerParams(
            dimension_semantics=("parallel",)
        ),
    )(x, offs, Wt, b2)


def kernel(category, brand, store, emb_category, emb_brand, emb_store, W, b):
    ci = category.astype(jnp.int32)
    bi = brand.astype(jnp.int32)
    si = store.astype(jnp.int32)
    p_cat = _tc_pack(emb_category.T)
    p_brand = emb_brand.reshape(-1, 4 * DIM)
    p_store = emb_store.reshape(-1, 4 * DIM)
    x = _sc_gather3(ci >> 2, bi >> 2, si >> 2, p_cat, p_brand, p_store)
    offs = jnp.stack([ci & 3, bi & 3, si & 3, jnp.zeros_like(ci)], axis=1)
    Wt = W.T
    b2 = b.reshape(1, OUT_DIM)
    return _tc_project(x, offs, Wt, b2)
